# baseline (speedup 1.0000x reference)
import functools

import jax
import jax.numpy as jnp
from jax import lax
from jax.experimental import pallas as pl
from jax.experimental.pallas import tpu as pltpu
from jax.experimental.pallas import tpu_sc as plsc

B, C, H, W = 8, 96, 224, 224
HW = H * W

HB = 32
NH = H // HB
SC_B = 1
TC_B = B - SC_B

NC, NS, L = 2, 16, 16
NW = NC * NS
SLAB_H = 8
SLABS_PER_IMG = H // SLAB_H
NSLAB = SC_B * SLABS_PER_IMG
NROUND = -(-NSLAB // NW)
CCH = 16
NCC = C // CCH
PIX_SLAB = SLAB_H * W
WV = W // L


def _tc_body(l_ref, lp_ref, cond_ref, z_ref, out_ref):
    lb = l_ref[0]
    lpb = lp_ref[0]
    g = jnp.zeros((HB, W), jnp.float32)
    bad = jnp.zeros((HB, W), jnp.float32)
    for c in range(C):
        zc = z_ref[0, c]
        g = jnp.where(lb == c, zc, g)
        bad = jnp.where(lpb == c, zc, bad)
    out_ref[pl.program_id(0), pl.program_id(1)] = jnp.sum(
        (g - bad) * cond_ref[0]
    )


def _sc_body(z_hbm, l_hbm, lp_hbm, cond_hbm, out_hbm,
             zb0, zb1, l_v, lp_v, cnd_v, g_v, b_v, acc_v, sem0, sem1):
    wid = lax.axis_index("s") * NC + lax.axis_index("c")
    acc_v[...] = jnp.zeros((L,), jnp.float32)

    for r in range(NROUND):
        s = wid + r * NW

        @pl.when(s < NSLAB)
        def _round():
            b = s // SLABS_PER_IMG
            h0 = (s % SLABS_PER_IMG) * SLAB_H
            pix0 = b * HW + h0 * W
            pltpu.sync_copy(l_hbm.at[pl.ds(pix0, PIX_SLAB)], l_v)
            pltpu.sync_copy(lp_hbm.at[pl.ds(pix0, PIX_SLAB)], lp_v)
            pltpu.sync_copy(cond_hbm.at[pl.ds(pix0, PIX_SLAB)], cnd_v)

            bufs = (zb0, zb1)
            sems = (sem0, sem1)
            handles = [None] * NCC
            handles[0] = pltpu.async_copy(
                z_hbm.at[b, pl.ds(0, CCH), pl.ds(h0, SLAB_H), :],
                zb0, sem0)

            for cc in range(NCC):
                if cc + 1 < NCC:
                    handles[cc + 1] = pltpu.async_copy(
                        z_hbm.at[b, pl.ds((cc + 1) * CCH, CCH),
                                 pl.ds(h0, SLAB_H), :],
                        bufs[(cc + 1) % 2], sems[(cc + 1) % 2])
                handles[cc].wait()
                zb = bufs[cc % 2]

                def row_loop(rr, acc, cc=cc, zb=zb):
                    def w_loop(wv, acc):
                        off = rr * W + wv * L
                        lv = l_v[pl.ds(off, L)]
                        lpv = lp_v[pl.ds(off, L)]
                        if cc == 0:
                            gv = jnp.zeros((L,), jnp.float32)
                            bv = jnp.zeros((L,), jnp.float32)
                        else:
                            gv = g_v[pl.ds(off, L)]
                            bv = b_v[pl.ds(off, L)]
                        for cl in range(CCH):
                            c = cc * CCH + cl
                            zv = zb[cl, rr, pl.ds(wv * L, L)]
                            gv = jnp.where(lv == c, zv, gv)
                            bv = jnp.where(lpv == c, zv, bv)
                        if cc == NCC - 1:
                            cv = cnd_v[pl.ds(off, L)]
                            acc = acc + (gv - bv) * cv
                        else:
                            g_v[pl.ds(off, L)] = gv
                            b_v[pl.ds(off, L)] = bv
                        return acc

                    return lax.fori_loop(0, WV, w_loop, acc)

                def slab_chunk(acc, cc=cc, zb=zb):
                    def r_loop(rr_, acc_):
                        return row_loop(rr_, acc_)
                    return lax.fori_loop(0, SLAB_H, r_loop, acc)

                acc_v[...] = slab_chunk(acc_v[...])

    pltpu.sync_copy(acc_v, out_hbm.at[wid])


@jax.jit
def _loss(z, l, lp, cond, l_sc, lp_sc, cond_sc):
    sc_partials = pl.kernel(
        _sc_body,
        out_type=jax.ShapeDtypeStruct((NW, L), jnp.float32),
        mesh=plsc.VectorSubcoreMesh(core_axis_name="c", subcore_axis_name="s"),
        scratch_types=[
            pltpu.VMEM((CCH, SLAB_H, W), jnp.float32),
            pltpu.VMEM((CCH, SLAB_H, W), jnp.float32),
            pltpu.VMEM((PIX_SLAB,), jnp.int32),
            pltpu.VMEM((PIX_SLAB,), jnp.int32),
            pltpu.VMEM((PIX_SLAB,), jnp.float32),
            pltpu.VMEM((PIX_SLAB,), jnp.float32),
            pltpu.VMEM((PIX_SLAB,), jnp.float32),
            pltpu.VMEM((L,), jnp.float32),
            pltpu.SemaphoreType.DMA,
            pltpu.SemaphoreType.DMA,
        ],
    )(z, l_sc, lp_sc, cond_sc)

    tc_partials = pl.pallas_call(
        _tc_body,
        grid=(TC_B, NH),
        in_specs=[
            pl.BlockSpec((1, HB, W), lambda b, j: (b + SC_B, j, 0)),
            pl.BlockSpec((1, HB, W), lambda b, j: (b + SC_B, j, 0)),
            pl.BlockSpec((1, HB, W), lambda b, j: (b + SC_B, j, 0)),
            pl.BlockSpec((1, C, HB, W), lambda b, j: (b + SC_B, 0, j, 0)),
        ],
        out_specs=pl.BlockSpec(
            (TC_B, NH), lambda b, j: (0, 0), memory_space=pltpu.SMEM
        ),
        out_shape=jax.ShapeDtypeStruct((TC_B, NH), jnp.float32),
        compiler_params=pltpu.CompilerParams(
            dimension_semantics=("arbitrary", "arbitrary"),
        ),
    )(l, lp, cond, z)

    return jnp.sum(tc_partials) + jnp.sum(sc_partials)


def kernel(z, condition, l, l_prime):
    l = l.astype(jnp.int32)
    lp = l_prime.astype(jnp.int32)
    cond = condition.astype(jnp.float32)
    l_sc = l[:SC_B].reshape(-1)
    lp_sc = lp[:SC_B].reshape(-1)
    cond_sc = cond[:SC_B].reshape(-1)
    return _loss(z, l, lp, cond, l_sc, lp_sc, cond_sc)


# speedup vs baseline: 1.1044x; 1.1044x over previous
import functools

import jax
import jax.numpy as jnp
from jax import lax
from jax.experimental import pallas as pl
from jax.experimental.pallas import tpu as pltpu
from jax.experimental.pallas import tpu_sc as plsc

B, C, H, W = 8, 96, 224, 224
HW = H * W

HB = 32
NH = H // HB
SC_B = 2
TC_B = B - SC_B

NC, NS, L = 2, 16, 16
NW = NC * NS
SLAB_H = 8
SLABS_PER_IMG = H // SLAB_H
NSLAB = SC_B * SLABS_PER_IMG
NROUND = -(-NSLAB // NW)
CCH = 16
NCC = C // CCH
WV = W // L


def _tc_body(l_ref, lp_ref, cond_ref, z_ref, out_ref):
    lb = l_ref[0]
    lpb = lp_ref[0]
    g = jnp.zeros((HB, W), jnp.float32)
    bad = jnp.zeros((HB, W), jnp.float32)
    for c in range(C):
        zc = z_ref[0, c]
        g = jnp.where(lb == c, zc, g)
        bad = jnp.where(lpb == c, zc, bad)
    part = jnp.sum((g - bad) * cond_ref[0])

    @pl.when(pl.program_id(0) + pl.program_id(1) == 0)
    def _init():
        out_ref[0] = jnp.float32(0.0)

    out_ref[0] += part


def _sc_body(z_hbm, l_hbm, lp_hbm, cond_hbm, out_hbm,
             zb0, zb1, l_v, lp_v, cnd_v, g_v, b_v, acc_v, sem0, sem1):
    wid = lax.axis_index("s") * NC + lax.axis_index("c")
    acc_v[...] = jnp.zeros((L,), jnp.float32)

    for r in range(NROUND):
        s = wid + r * NW

        @pl.when(s < NSLAB)
        def _round():
            b = s // SLABS_PER_IMG
            h0 = (s % SLABS_PER_IMG) * SLAB_H
            pltpu.sync_copy(l_hbm.at[b, pl.ds(h0, SLAB_H), :], l_v)
            pltpu.sync_copy(lp_hbm.at[b, pl.ds(h0, SLAB_H), :], lp_v)
            pltpu.sync_copy(cond_hbm.at[b, pl.ds(h0, SLAB_H), :], cnd_v)

            bufs = (zb0, zb1)
            sems = (sem0, sem1)
            handles = [None] * NCC
            handles[0] = pltpu.async_copy(
                z_hbm.at[b, pl.ds(0, CCH), pl.ds(h0, SLAB_H), :],
                zb0, sem0)

            for cc in range(NCC):
                if cc + 1 < NCC:
                    handles[cc + 1] = pltpu.async_copy(
                        z_hbm.at[b, pl.ds((cc + 1) * CCH, CCH),
                                 pl.ds(h0, SLAB_H), :],
                        bufs[(cc + 1) % 2], sems[(cc + 1) % 2])
                handles[cc].wait()
                zb = bufs[cc % 2]

                def row_loop(rr, acc, cc=cc, zb=zb):
                    def w_loop(wv, acc):
                        wv16 = wv * L
                        lv = l_v[rr, pl.ds(wv16, L)]
                        lpv = lp_v[rr, pl.ds(wv16, L)]
                        off = rr * W + wv16
                        if cc == 0:
                            gv = jnp.zeros((L,), jnp.float32)
                            bv = jnp.zeros((L,), jnp.float32)
                        else:
                            gv = g_v[pl.ds(off, L)]
                            bv = b_v[pl.ds(off, L)]
                        for cl in range(CCH):
                            c = cc * CCH + cl
                            zv = zb[cl, rr, pl.ds(wv16, L)]
                            gv = jnp.where(lv == c, zv, gv)
                            bv = jnp.where(lpv == c, zv, bv)
                        if cc == NCC - 1:
                            cv = cnd_v[rr, pl.ds(wv16, L)]
                            acc = acc + (gv - bv) * cv
                        else:
                            g_v[pl.ds(off, L)] = gv
                            b_v[pl.ds(off, L)] = bv
                        return acc

                    return lax.fori_loop(0, WV, w_loop, acc)

                def slab_chunk(acc, cc=cc, zb=zb):
                    def r_loop(rr_, acc_):
                        return row_loop(rr_, acc_)
                    return lax.fori_loop(0, SLAB_H, r_loop, acc)

                acc_v[...] = slab_chunk(acc_v[...])

    pltpu.sync_copy(acc_v, out_hbm.at[wid])


@jax.jit
def _loss(z, l, lp, cond):
    sc_partials = pl.kernel(
        _sc_body,
        out_type=jax.ShapeDtypeStruct((NW, L), jnp.float32),
        mesh=plsc.VectorSubcoreMesh(core_axis_name="c", subcore_axis_name="s"),
        scratch_types=[
            pltpu.VMEM((CCH, SLAB_H, W), jnp.float32),
            pltpu.VMEM((CCH, SLAB_H, W), jnp.float32),
            pltpu.VMEM((SLAB_H, W), jnp.int32),
            pltpu.VMEM((SLAB_H, W), jnp.int32),
            pltpu.VMEM((SLAB_H, W), jnp.float32),
            pltpu.VMEM((SLAB_H * W,), jnp.float32),
            pltpu.VMEM((SLAB_H * W,), jnp.float32),
            pltpu.VMEM((L,), jnp.float32),
            pltpu.SemaphoreType.DMA,
            pltpu.SemaphoreType.DMA,
        ],
    )(z, l, lp, cond)

    tc_partial = pl.pallas_call(
        _tc_body,
        grid=(TC_B, NH),
        in_specs=[
            pl.BlockSpec((1, HB, W), lambda b, j: (b + SC_B, j, 0)),
            pl.BlockSpec((1, HB, W), lambda b, j: (b + SC_B, j, 0)),
            pl.BlockSpec((1, HB, W), lambda b, j: (b + SC_B, j, 0)),
            pl.BlockSpec((1, C, HB, W), lambda b, j: (b + SC_B, 0, j, 0)),
        ],
        out_specs=pl.BlockSpec(
            (1,), lambda b, j: (0,), memory_space=pltpu.SMEM
        ),
        out_shape=jax.ShapeDtypeStruct((1,), jnp.float32),
        compiler_params=pltpu.CompilerParams(
            dimension_semantics=("arbitrary", "arbitrary"),
        ),
    )(l, lp, cond, z)

    return tc_partial[0] + jnp.sum(sc_partials)


def kernel(z, condition, l, l_prime):
    return _loss(
        z,
        l.astype(jnp.int32),
        l_prime.astype(jnp.int32),
        condition.astype(jnp.float32),
    )
